# baseline (speedup 1.0000x reference)
import functools

import jax
import jax.numpy as jnp
from jax import lax
from jax.experimental import pallas as pl
from jax.experimental.pallas import tpu as pltpu
from jax.experimental.pallas import tpu_sc as plsc

T = 256
FB = 1024


def _routing_metadata(expert_indices, expert_weights, E, T, G_MAX, T_MAX, n_ff):
    P = expert_indices.size
    e = expert_indices.reshape(P).astype(jnp.int32)
    w = expert_weights.reshape(P)
    topk = expert_indices.shape[-1]
    tok = (jnp.arange(P, dtype=jnp.int32) // topk).astype(jnp.int32)

    onehot = (e[:, None] == jnp.arange(E, dtype=jnp.int32)[None, :]).astype(jnp.int32)
    counts = jnp.sum(onehot, axis=0)
    ranks = jnp.cumsum(onehot, axis=0) - onehot
    rank = jnp.take_along_axis(ranks, e[:, None], axis=1)[:, 0]

    tiles_per_e = (counts + T - 1) // T
    tile_start = jnp.concatenate(
        [jnp.zeros((1,), jnp.int32), jnp.cumsum(tiles_per_e).astype(jnp.int32)])
    padded_start = tile_start[:-1] * T
    pos = padded_start[e] + rank

    del w

    J = n_ff
    n_e = tiles_per_e.astype(jnp.int32)
    N_STEPS = G_MAX * J
    n_steps = tile_start[E] * J
    k_ids = jnp.arange(N_STEPS, dtype=jnp.int32)
    live = k_ids < n_steps
    kc = jnp.minimum(k_ids, jnp.maximum(n_steps - 1, 0))
    estep = J * tile_start
    sched_e = jnp.sum((kc[:, None] >= estep[1:][None, :]).astype(jnp.int32),
                      axis=1)
    ne_k = jnp.maximum(n_e[sched_e], 1)
    r = kc - estep[sched_e]
    sched_j = r // ne_k
    sched_t = r - sched_j * ne_k
    sched_g = tile_start[sched_e] + sched_t
    sched_valid = live.astype(jnp.int32)
    grp_first = jnp.concatenate(
        [jnp.ones((1,), jnp.int32),
         ((sched_e[1:] != sched_e[:-1]) | (sched_j[1:] != sched_j[:-1])
          ).astype(jnp.int32) * sched_valid[1:]])
    gid = jnp.cumsum(grp_first) - 1
    nz = (n_e > 0).astype(jnp.int32)
    n_groups = jnp.sum(nz) * J
    nonempty = jnp.zeros((E,), jnp.int32).at[
        jnp.where(nz == 1, jnp.cumsum(nz) - 1, E)].set(
            jnp.arange(E, dtype=jnp.int32), mode="drop")
    nxt = jnp.minimum(gid + 1, jnp.maximum(n_groups - 1, 0))
    pf_e = nonempty[nxt // J]
    pf_j = (nxt % J).astype(jnp.int32)
    pf_valid = ((gid + 1) < n_groups).astype(jnp.int32) * grp_first
    cur_buf = (gid % 2).astype(jnp.int32)
    sched = jnp.stack([sched_e, sched_j, sched_t, sched_g, sched_valid,
                       grp_first, cur_buf, pf_valid, pf_e, pf_j])
    return tok, pos, sched


def _make_sc_dispatch(P, P_MAX, H):
    info = plsc.get_sparse_core_info()
    NW = info.num_cores * info.num_subcores
    b_per_w = P // NW
    L = info.num_lanes
    CHUNK = 32
    nch = b_per_w // CHUNK
    NBUF = 3
    mesh = plsc.VectorSubcoreMesh(core_axis_name="c", subcore_axis_name="s")

    @functools.partial(
        pl.kernel,
        out_type=jax.ShapeDtypeStruct((P_MAX, H), jnp.float32),
        mesh=mesh,
        scratch_types=[
            [pltpu.VMEM((CHUNK,), jnp.int32) for _ in range(nch)],
            [pltpu.VMEM((CHUNK,), jnp.int32) for _ in range(nch)],
            pltpu.VMEM((NBUF, CHUNK, H), jnp.float32),
            [pltpu.SemaphoreType.DMA] * NBUF,
            [pltpu.SemaphoreType.DMA] * NBUF,
        ],
    )
    def dispatch_k(x_hbm, tok_hbm, pos_hbm, xg_hbm,
                   tok_vs, pos_vs, rows_v, gsems, ssems):
        wid = lax.axis_index("s") * info.num_cores + lax.axis_index("c")
        base = wid * b_per_w

        def start_gather(i):
            return pltpu.async_copy(
                x_hbm.at[tok_vs[i]], rows_v.at[i % NBUF], gsems[i % NBUF])

        def start_scatter(i):
            return pltpu.async_copy(
                rows_v.at[i % NBUF], xg_hbm.at[pos_vs[i]], ssems[i % NBUF])

        for i in range(nch):
            off = base + i * CHUNK
            pltpu.sync_copy(tok_hbm.at[pl.ds(off, CHUNK)], tok_vs[i])
            pltpu.sync_copy(pos_hbm.at[pl.ds(off, CHUNK)], pos_vs[i])
        gops = {}
        sops = {}
        for i in range(min(NBUF - 1, nch)):
            gops[i] = start_gather(i)
        for i in range(nch):
            gops[i].wait()
            sops[i] = start_scatter(i)
            k = i + NBUF - 1
            if k < nch:
                if k - NBUF >= 0:
                    sops[k - NBUF].wait()
                gops[k] = start_gather(k)
        for i in range(max(0, nch - NBUF), nch):
            if i in sops:
                sops[i].wait()

    return dispatch_k


def _make_sc_combine(S, H, P_MAX):
    info = plsc.get_sparse_core_info()
    NW = info.num_cores * info.num_subcores
    t_per_w = S // NW
    CH = 32
    n_ch = t_per_w // CH
    L = info.num_lanes
    mesh = plsc.VectorSubcoreMesh(core_axis_name="c", subcore_axis_name="s")

    @functools.partial(
        pl.kernel,
        out_type=jax.ShapeDtypeStruct((S, H), jnp.float32),
        mesh=mesh,
        scratch_types=[
            pltpu.VMEM((CH,), jnp.int32),
            pltpu.VMEM((CH,), jnp.int32),
            pltpu.VMEM((CH, H), jnp.float32),
            pltpu.VMEM((CH, H), jnp.float32),
            pltpu.SemaphoreType.DMA,
            pltpu.SemaphoreType.DMA,
        ],
    )
    def combine_k(yg_hbm, pos0_hbm, pos1_hbm, out_hbm,
                  i0_v, i1_v, a_v, b_v, sem0, sem1):
        wid = lax.axis_index("s") * info.num_cores + lax.axis_index("c")
        base = wid * t_per_w
        for c in range(n_ch):
            cbase = base + c * CH
            pltpu.sync_copy(pos0_hbm.at[pl.ds(cbase, CH)], i0_v)
            pltpu.sync_copy(pos1_hbm.at[pl.ds(cbase, CH)], i1_v)
            cp0 = pltpu.async_copy(yg_hbm.at[i0_v], a_v, sem0)
            cp1 = pltpu.async_copy(yg_hbm.at[i1_v], b_v, sem1)
            cp0.wait()
            cp1.wait()

            def add_row(r, _):
                def add_vec(v, _):
                    sl = pl.ds(v * L, L)
                    a_v[r, sl] = a_v[r, sl] + b_v[r, sl]
                    return 0
                return lax.fori_loop(0, H // L, add_vec, 0)

            lax.fori_loop(0, CH, add_row, 0)
            pltpu.sync_copy(a_v, out_hbm.at[pl.ds(cbase, CH)])

    return combine_k


def _ffn_body(sched_ref, xg_ref, gw_hbm, uw_hbm, dw_hbm, w_ref, out_ref,
              acc_ref, gsc, usc, dsc, gsem, usem, dsem,
              *, n_ff_blocks, fb):
    k = pl.program_id(0)
    j = sched_ref[1, k]
    t = sched_ref[2, k]
    buf = sched_ref[6, k]

    def wcopies(slot, pe, pj):
        return (
            pltpu.make_async_copy(
                gw_hbm.at[pe, pl.ds(pj * fb, fb), :], gsc.at[slot],
                gsem.at[slot]),
            pltpu.make_async_copy(
                uw_hbm.at[pe, pl.ds(pj * fb, fb), :], usc.at[slot],
                usem.at[slot]),
            pltpu.make_async_copy(
                dw_hbm.at[pe, :, pl.ds(pj * fb, fb)], dsc.at[slot],
                dsem.at[slot]),
        )

    @pl.when(sched_ref[5, k] == 1)
    def _():
        @pl.when(k == 0)
        def _():
            for cp in wcopies(buf, sched_ref[0, k], sched_ref[1, k]):
                cp.start()

        for cp in wcopies(buf, sched_ref[0, k], sched_ref[1, k]):
            cp.wait()

        @pl.when(sched_ref[7, k] == 1)
        def _():
            for cp in wcopies(1 - buf, sched_ref[8, k], sched_ref[9, k]):
                cp.start()

    @pl.when(sched_ref[4, k] == 1)
    def _():
        x = xg_ref[...]
        gate = lax.dot_general(
            x, gsc.at[buf][...], (((1,), (1,)), ((), ())),
            preferred_element_type=jnp.float32)
        up = lax.dot_general(
            x, usc.at[buf][...], (((1,), (1,)), ((), ())),
            preferred_element_type=jnp.float32)
        inter = (gate * jax.nn.sigmoid(gate)) * up
        part = lax.dot_general(
            inter, dsc.at[buf][...], (((1,), (1,)), ((), ())),
            preferred_element_type=jnp.float32)

        @pl.when(j == 0)
        def _():
            acc_ref[t] = part

        @pl.when((j > 0) & (j < n_ff_blocks - 1))
        def _():
            acc_ref[t] = acc_ref[t] + part

        @pl.when(j == n_ff_blocks - 1)
        def _():
            out_ref[...] = (acc_ref[t] + part) * w_ref[...]


def kernel(x, expert_indices, expert_weights, gate_proj, up_proj, down_proj):
    b, s, h = x.shape
    E, FF, _ = gate_proj.shape
    topk = expert_indices.shape[-1]
    P = b * s * topk
    G_MAX = P // T + (E - 1)
    P_MAX = G_MAX * T
    J = FF // FB

    T_MAX = -(-P // T)
    N_STEPS = G_MAX * J
    x_flat = x.reshape(b * s, h)
    tok, pos, sched = _routing_metadata(
        expert_indices, expert_weights, E, T, G_MAX, T_MAX, J)

    xg = _make_sc_dispatch(P, P_MAX, h)(x_flat, tok, pos.astype(jnp.int32))
    row_w = jnp.zeros((P_MAX,), jnp.float32).at[pos].set(
        expert_weights.reshape(P))

    def out_idx(k, sc):
        ok = (sc[1, k] == J - 1) & (sc[4, k] == 1)
        return (jnp.where(ok, sc[3, k], G_MAX), 0)

    grid_spec = pltpu.PrefetchScalarGridSpec(
        num_scalar_prefetch=1,
        grid=(N_STEPS,),
        in_specs=[
            pl.BlockSpec((T, h), lambda k, sc: (sc[3, k], 0)),
            pl.BlockSpec(memory_space=pl.ANY),
            pl.BlockSpec(memory_space=pl.ANY),
            pl.BlockSpec(memory_space=pl.ANY),
            pl.BlockSpec((T, 1), lambda k, sc: (sc[3, k], 0)),
        ],
        out_specs=pl.BlockSpec((T, h), out_idx),
        scratch_shapes=[
            pltpu.VMEM((T_MAX, T, h), jnp.float32),
            pltpu.VMEM((2, FB, h), jnp.float32),
            pltpu.VMEM((2, FB, h), jnp.float32),
            pltpu.VMEM((2, h, FB), jnp.float32),
            pltpu.SemaphoreType.DMA((2,)),
            pltpu.SemaphoreType.DMA((2,)),
            pltpu.SemaphoreType.DMA((2,)),
        ],
    )
    yg = pl.pallas_call(
        functools.partial(_ffn_body, n_ff_blocks=J, fb=FB),
        grid_spec=grid_spec,
        out_shape=jax.ShapeDtypeStruct(((G_MAX + 1) * T, h), jnp.float32),
        compiler_params=pltpu.CompilerParams(
            dimension_semantics=("arbitrary",)),
    )(sched, xg, gate_proj, up_proj, down_proj, row_w.reshape(P_MAX, 1))

    pos2 = pos.reshape(b * s, topk)
    out = _make_sc_combine(b * s, h, (G_MAX + 1) * T)(
        yg, pos2[:, 0].astype(jnp.int32), pos2[:, 1].astype(jnp.int32))
    return out.reshape(b, s, h)


# speedup vs baseline: 1.0397x; 1.0397x over previous
import functools

import jax
import jax.numpy as jnp
from jax import lax
from jax.experimental import pallas as pl
from jax.experimental.pallas import tpu as pltpu
from jax.experimental.pallas import tpu_sc as plsc

T = 256
FB = 1024


def _routing_metadata(expert_indices, expert_weights, E, T, G_MAX, T_MAX, n_ff):
    P = expert_indices.size
    e = expert_indices.reshape(P).astype(jnp.int32)
    w = expert_weights.reshape(P)
    topk = expert_indices.shape[-1]
    tok = (jnp.arange(P, dtype=jnp.int32) // topk).astype(jnp.int32)

    onehot = (e[:, None] == jnp.arange(E, dtype=jnp.int32)[None, :]).astype(jnp.int32)
    counts = jnp.sum(onehot, axis=0)
    ranks = jnp.cumsum(onehot, axis=0) - onehot
    rank = jnp.take_along_axis(ranks, e[:, None], axis=1)[:, 0]

    tiles_per_e = (counts + T - 1) // T
    tile_start = jnp.concatenate(
        [jnp.zeros((1,), jnp.int32), jnp.cumsum(tiles_per_e).astype(jnp.int32)])
    padded_start = tile_start[:-1] * T
    pos = padded_start[e] + rank

    del w

    J = n_ff
    n_e = tiles_per_e.astype(jnp.int32)
    N_STEPS = G_MAX * J
    n_steps = tile_start[E] * J
    k_ids = jnp.arange(N_STEPS, dtype=jnp.int32)
    live = k_ids < n_steps
    kc = jnp.minimum(k_ids, jnp.maximum(n_steps - 1, 0))
    estep = J * tile_start
    sched_e = jnp.sum((kc[:, None] >= estep[1:][None, :]).astype(jnp.int32),
                      axis=1)
    ne_k = jnp.maximum(n_e[sched_e], 1)
    r = kc - estep[sched_e]
    sched_j = r // ne_k
    sched_t = r - sched_j * ne_k
    sched_g = tile_start[sched_e] + sched_t
    sched_valid = live.astype(jnp.int32)
    grp_first = jnp.concatenate(
        [jnp.ones((1,), jnp.int32),
         ((sched_e[1:] != sched_e[:-1]) | (sched_j[1:] != sched_j[:-1])
          ).astype(jnp.int32) * sched_valid[1:]])
    gid = jnp.cumsum(grp_first) - 1
    nz = (n_e > 0).astype(jnp.int32)
    n_groups = jnp.sum(nz) * J
    nonempty = jnp.zeros((E,), jnp.int32).at[
        jnp.where(nz == 1, jnp.cumsum(nz) - 1, E)].set(
            jnp.arange(E, dtype=jnp.int32), mode="drop")
    nxt = jnp.minimum(gid + 1, jnp.maximum(n_groups - 1, 0))
    pf_e = nonempty[nxt // J]
    pf_j = (nxt % J).astype(jnp.int32)
    pf_valid = ((gid + 1) < n_groups).astype(jnp.int32) * grp_first
    cur_buf = (gid % 2).astype(jnp.int32)
    sched = jnp.stack([sched_e, sched_j, sched_t, sched_g, sched_valid,
                       grp_first, cur_buf, pf_valid, pf_e, pf_j])
    return tok, pos, sched


def _make_sc_dispatch(P, P_MAX, H):
    info = plsc.get_sparse_core_info()
    NW = info.num_cores * info.num_subcores
    b_per_w = P // NW
    L = info.num_lanes
    CHUNK = 32
    nch = b_per_w // CHUNK
    NBUF = 3
    mesh = plsc.VectorSubcoreMesh(core_axis_name="c", subcore_axis_name="s")

    @functools.partial(
        pl.kernel,
        out_type=jax.ShapeDtypeStruct((P_MAX, H), jnp.float32),
        mesh=mesh,
        scratch_types=[
            [pltpu.VMEM((CHUNK,), jnp.int32) for _ in range(nch)],
            [pltpu.VMEM((CHUNK,), jnp.int32) for _ in range(nch)],
            pltpu.VMEM((NBUF, CHUNK, H), jnp.float32),
            [pltpu.SemaphoreType.DMA] * NBUF,
            [pltpu.SemaphoreType.DMA] * NBUF,
        ],
    )
    def dispatch_k(x_hbm, tok_hbm, pos_hbm, xg_hbm,
                   tok_vs, pos_vs, rows_v, gsems, ssems):
        wid = lax.axis_index("s") * info.num_cores + lax.axis_index("c")
        base = wid * b_per_w

        def start_gather(i):
            return pltpu.async_copy(
                x_hbm.at[tok_vs[i]], rows_v.at[i % NBUF], gsems[i % NBUF])

        def start_scatter(i):
            return pltpu.async_copy(
                rows_v.at[i % NBUF], xg_hbm.at[pos_vs[i]], ssems[i % NBUF])

        for i in range(nch):
            off = base + i * CHUNK
            pltpu.sync_copy(tok_hbm.at[pl.ds(off, CHUNK)], tok_vs[i])
            pltpu.sync_copy(pos_hbm.at[pl.ds(off, CHUNK)], pos_vs[i])
        gops = {}
        sops = {}
        for i in range(min(NBUF - 1, nch)):
            gops[i] = start_gather(i)
        for i in range(nch):
            gops[i].wait()
            sops[i] = start_scatter(i)
            k = i + NBUF - 1
            if k < nch:
                if k - NBUF >= 0:
                    sops[k - NBUF].wait()
                gops[k] = start_gather(k)
        for i in range(max(0, nch - NBUF), nch):
            if i in sops:
                sops[i].wait()

    return dispatch_k


def _make_sc_combine(S, H, P_MAX):
    info = plsc.get_sparse_core_info()
    NW = info.num_cores * info.num_subcores
    t_per_w = S // NW
    CH = 32
    n_ch = t_per_w // CH
    L = info.num_lanes
    mesh = plsc.VectorSubcoreMesh(core_axis_name="c", subcore_axis_name="s")

    @functools.partial(
        pl.kernel,
        out_type=jax.ShapeDtypeStruct((S, H), jnp.float32),
        mesh=mesh,
        scratch_types=[
            pltpu.VMEM((CH,), jnp.int32),
            pltpu.VMEM((CH,), jnp.int32),
            pltpu.VMEM((CH, 16), jnp.float32),
            pltpu.VMEM((CH, 16), jnp.float32),
            pltpu.VMEM((CH, H), jnp.float32),
            pltpu.VMEM((CH, H), jnp.float32),
            pltpu.SemaphoreType.DMA,
            pltpu.SemaphoreType.DMA,
        ],
    )
    def combine_k(yg_hbm, pos0_hbm, pos1_hbm, w0_hbm, w1_hbm, out_hbm,
                  i0_v, i1_v, w0_v, w1_v, a_v, b_v, sem0, sem1):
        wid = lax.axis_index("s") * info.num_cores + lax.axis_index("c")
        base = wid * t_per_w
        for c in range(n_ch):
            cbase = base + c * CH
            pltpu.sync_copy(pos0_hbm.at[pl.ds(cbase, CH)], i0_v)
            pltpu.sync_copy(pos1_hbm.at[pl.ds(cbase, CH)], i1_v)
            cp0 = pltpu.async_copy(yg_hbm.at[i0_v], a_v, sem0)
            cp1 = pltpu.async_copy(yg_hbm.at[i1_v], b_v, sem1)
            pltpu.sync_copy(w0_hbm.at[pl.ds(cbase, CH)], w0_v)
            pltpu.sync_copy(w1_hbm.at[pl.ds(cbase, CH)], w1_v)
            cp0.wait()
            cp1.wait()

            def add_row(r, _):
                w0b = w0_v[r, :]
                w1b = w1_v[r, :]

                def add_vec(v, _):
                    sl = pl.ds(v * L, L)
                    a_v[r, sl] = a_v[r, sl] * w0b + b_v[r, sl] * w1b
                    return 0
                return lax.fori_loop(0, H // L, add_vec, 0)

            lax.fori_loop(0, CH, add_row, 0)
            pltpu.sync_copy(a_v, out_hbm.at[pl.ds(cbase, CH)])

    return combine_k


def _ffn_body(sched_ref, xg_ref, gw_hbm, uw_hbm, dw_hbm, out_ref,
              acc_ref, gsc, usc, dsc, gsem, usem, dsem,
              *, n_ff_blocks, fb):
    k = pl.program_id(0)
    j = sched_ref[1, k]
    t = sched_ref[2, k]
    buf = sched_ref[6, k]

    def wcopies(slot, pe, pj):
        return (
            pltpu.make_async_copy(
                gw_hbm.at[pe, pl.ds(pj * fb, fb), :], gsc.at[slot],
                gsem.at[slot]),
            pltpu.make_async_copy(
                uw_hbm.at[pe, pl.ds(pj * fb, fb), :], usc.at[slot],
                usem.at[slot]),
            pltpu.make_async_copy(
                dw_hbm.at[pe, :, pl.ds(pj * fb, fb)], dsc.at[slot],
                dsem.at[slot]),
        )

    @pl.when(sched_ref[5, k] == 1)
    def _():
        @pl.when(k == 0)
        def _():
            for cp in wcopies(buf, sched_ref[0, k], sched_ref[1, k]):
                cp.start()

        for cp in wcopies(buf, sched_ref[0, k], sched_ref[1, k]):
            cp.wait()

        @pl.when(sched_ref[7, k] == 1)
        def _():
            for cp in wcopies(1 - buf, sched_ref[8, k], sched_ref[9, k]):
                cp.start()

    @pl.when(sched_ref[4, k] == 1)
    def _():
        x = xg_ref[...]
        gate = lax.dot_general(
            x, gsc.at[buf][...], (((1,), (1,)), ((), ())),
            preferred_element_type=jnp.float32)
        up = lax.dot_general(
            x, usc.at[buf][...], (((1,), (1,)), ((), ())),
            preferred_element_type=jnp.float32)
        inter = (gate * jax.nn.sigmoid(gate)) * up
        part = lax.dot_general(
            inter, dsc.at[buf][...], (((1,), (1,)), ((), ())),
            preferred_element_type=jnp.float32)

        @pl.when(j == 0)
        def _():
            acc_ref[t] = part

        @pl.when((j > 0) & (j < n_ff_blocks - 1))
        def _():
            acc_ref[t] = acc_ref[t] + part

        @pl.when(j == n_ff_blocks - 1)
        def _():
            out_ref[...] = acc_ref[t] + part


def kernel(x, expert_indices, expert_weights, gate_proj, up_proj, down_proj):
    b, s, h = x.shape
    E, FF, _ = gate_proj.shape
    topk = expert_indices.shape[-1]
    P = b * s * topk
    G_MAX = P // T + (E - 1)
    P_MAX = G_MAX * T
    J = FF // FB

    T_MAX = -(-P // T)
    N_STEPS = G_MAX * J
    x_flat = x.reshape(b * s, h)
    tok, pos, sched = _routing_metadata(
        expert_indices, expert_weights, E, T, G_MAX, T_MAX, J)

    xg = _make_sc_dispatch(P, P_MAX, h)(x_flat, tok, pos.astype(jnp.int32))

    def out_idx(k, sc):
        ok = (sc[1, k] == J - 1) & (sc[4, k] == 1)
        return (jnp.where(ok, sc[3, k], G_MAX), 0)

    grid_spec = pltpu.PrefetchScalarGridSpec(
        num_scalar_prefetch=1,
        grid=(N_STEPS,),
        in_specs=[
            pl.BlockSpec((T, h), lambda k, sc: (sc[3, k], 0)),
            pl.BlockSpec(memory_space=pl.ANY),
            pl.BlockSpec(memory_space=pl.ANY),
            pl.BlockSpec(memory_space=pl.ANY),
        ],
        out_specs=pl.BlockSpec((T, h), out_idx),
        scratch_shapes=[
            pltpu.VMEM((T_MAX, T, h), jnp.float32),
            pltpu.VMEM((2, FB, h), jnp.float32),
            pltpu.VMEM((2, FB, h), jnp.float32),
            pltpu.VMEM((2, h, FB), jnp.float32),
            pltpu.SemaphoreType.DMA((2,)),
            pltpu.SemaphoreType.DMA((2,)),
            pltpu.SemaphoreType.DMA((2,)),
        ],
    )
    yg = pl.pallas_call(
        functools.partial(_ffn_body, n_ff_blocks=J, fb=FB),
        grid_spec=grid_spec,
        out_shape=jax.ShapeDtypeStruct(((G_MAX + 1) * T, h), jnp.float32),
        compiler_params=pltpu.CompilerParams(
            dimension_semantics=("arbitrary",)),
    )(sched, xg, gate_proj, up_proj, down_proj)

    pos2 = pos.reshape(b * s, topk)
    ew = expert_weights.reshape(b * s, topk)
    ones = jnp.ones((1, 16), jnp.float32)
    out = _make_sc_combine(b * s, h, (G_MAX + 1) * T)(
        yg, pos2[:, 0].astype(jnp.int32), pos2[:, 1].astype(jnp.int32),
        ew[:, 0:1] * ones, ew[:, 1:2] * ones)
    return out.reshape(b, s, h)


# speedup vs baseline: 1.0584x; 1.0180x over previous
import functools

import jax
import jax.numpy as jnp
from jax import lax
from jax.experimental import pallas as pl
from jax.experimental.pallas import tpu as pltpu
from jax.experimental.pallas import tpu_sc as plsc

T = 256
FB = 1024


def _routing_metadata(expert_indices, expert_weights, E, T, G_MAX, T_MAX, n_ff):
    P = expert_indices.size
    e = expert_indices.reshape(P).astype(jnp.int32)
    w = expert_weights.reshape(P)
    topk = expert_indices.shape[-1]
    tok = (jnp.arange(P, dtype=jnp.int32) // topk).astype(jnp.int32)

    onehot = (e[:, None] == jnp.arange(E, dtype=jnp.int32)[None, :]).astype(jnp.int32)
    counts = jnp.sum(onehot, axis=0)
    ranks = jnp.cumsum(onehot, axis=0) - onehot
    rank = jnp.take_along_axis(ranks, e[:, None], axis=1)[:, 0]

    tiles_per_e = (counts + T - 1) // T
    tile_start = jnp.concatenate(
        [jnp.zeros((1,), jnp.int32), jnp.cumsum(tiles_per_e).astype(jnp.int32)])
    padded_start = tile_start[:-1] * T
    pos = padded_start[e] + rank

    del w

    J = n_ff
    n_e = tiles_per_e.astype(jnp.int32)
    N_STEPS = G_MAX * J
    n_steps = tile_start[E] * J
    k_ids = jnp.arange(N_STEPS, dtype=jnp.int32)
    live = k_ids < n_steps
    kc = jnp.minimum(k_ids, jnp.maximum(n_steps - 1, 0))
    estep = J * tile_start
    sched_e = jnp.sum((kc[:, None] >= estep[1:][None, :]).astype(jnp.int32),
                      axis=1)
    ne_k = jnp.maximum(n_e[sched_e], 1)
    r = kc - estep[sched_e]
    sched_j = r // ne_k
    sched_t = r - sched_j * ne_k
    sched_g = tile_start[sched_e] + sched_t
    sched_valid = live.astype(jnp.int32)
    grp_first = jnp.concatenate(
        [jnp.ones((1,), jnp.int32),
         ((sched_e[1:] != sched_e[:-1]) | (sched_j[1:] != sched_j[:-1])
          ).astype(jnp.int32) * sched_valid[1:]])
    gid = jnp.cumsum(grp_first) - 1
    e_row = jnp.arange(E, dtype=jnp.int32)
    ne_next = jnp.min(
        jnp.where((e_row[None, :] > e_row[:, None]) & (n_e[None, :] > 0),
                  e_row[None, :], E), axis=1)
    last_j = sched_j == (J - 1)
    pf_e = jnp.where(last_j, ne_next[sched_e], sched_e)
    pf_j = jnp.where(last_j, 0, sched_j + 1)
    pf_valid = grp_first * (1 - (last_j & (ne_next[sched_e] == E)
                                 ).astype(jnp.int32))
    pf_e = jnp.minimum(pf_e, E - 1)
    cur_buf = (gid % 2).astype(jnp.int32)
    sched = jnp.stack([sched_e, sched_j, sched_t, sched_g, sched_valid,
                       grp_first, cur_buf, pf_valid, pf_e, pf_j])
    return tok, pos, sched


def _make_sc_dispatch(P, P_MAX, H):
    info = plsc.get_sparse_core_info()
    NW = info.num_cores * info.num_subcores
    b_per_w = P // NW
    L = info.num_lanes
    CHUNK = 32
    nch = b_per_w // CHUNK
    NBUF = 3
    mesh = plsc.VectorSubcoreMesh(core_axis_name="c", subcore_axis_name="s")

    @functools.partial(
        pl.kernel,
        out_type=jax.ShapeDtypeStruct((P_MAX, H), jnp.float32),
        mesh=mesh,
        scratch_types=[
            [pltpu.VMEM((CHUNK,), jnp.int32) for _ in range(nch)],
            [pltpu.VMEM((CHUNK,), jnp.int32) for _ in range(nch)],
            pltpu.VMEM((NBUF, CHUNK, H), jnp.float32),
            [pltpu.SemaphoreType.DMA] * NBUF,
            [pltpu.SemaphoreType.DMA] * NBUF,
        ],
    )
    def dispatch_k(x_hbm, tok_hbm, pos_hbm, xg_hbm,
                   tok_vs, pos_vs, rows_v, gsems, ssems):
        wid = lax.axis_index("s") * info.num_cores + lax.axis_index("c")
        base = wid * b_per_w

        def start_gather(i):
            return pltpu.async_copy(
                x_hbm.at[tok_vs[i]], rows_v.at[i % NBUF], gsems[i % NBUF])

        def start_scatter(i):
            return pltpu.async_copy(
                rows_v.at[i % NBUF], xg_hbm.at[pos_vs[i]], ssems[i % NBUF])

        for i in range(nch):
            off = base + i * CHUNK
            pltpu.sync_copy(tok_hbm.at[pl.ds(off, CHUNK)], tok_vs[i])
            pltpu.sync_copy(pos_hbm.at[pl.ds(off, CHUNK)], pos_vs[i])
        gops = {}
        sops = {}
        for i in range(min(NBUF - 1, nch)):
            gops[i] = start_gather(i)
        for i in range(nch):
            gops[i].wait()
            sops[i] = start_scatter(i)
            k = i + NBUF - 1
            if k < nch:
                if k - NBUF >= 0:
                    sops[k - NBUF].wait()
                gops[k] = start_gather(k)
        for i in range(max(0, nch - NBUF), nch):
            if i in sops:
                sops[i].wait()

    return dispatch_k


def _make_sc_combine(S, H, P_MAX):
    info = plsc.get_sparse_core_info()
    NW = info.num_cores * info.num_subcores
    t_per_w = S // NW
    CH = 16
    n_ch = t_per_w // CH
    L = info.num_lanes
    mesh = plsc.VectorSubcoreMesh(core_axis_name="c", subcore_axis_name="s")

    @functools.partial(
        pl.kernel,
        out_type=jax.ShapeDtypeStruct((S, H), jnp.float32),
        mesh=mesh,
        scratch_types=[
            [pltpu.VMEM((CH,), jnp.int32) for _ in range(n_ch)],
            [pltpu.VMEM((CH,), jnp.int32) for _ in range(n_ch)],
            pltpu.VMEM((t_per_w, 16), jnp.float32),
            pltpu.VMEM((t_per_w, 16), jnp.float32),
            pltpu.VMEM((2, CH, H), jnp.float32),
            pltpu.VMEM((2, CH, H), jnp.float32),
            [pltpu.SemaphoreType.DMA] * 2,
            [pltpu.SemaphoreType.DMA] * 2,
            [pltpu.SemaphoreType.DMA] * 2,
        ],
    )
    def combine_k(yg_hbm, pos0_hbm, pos1_hbm, w0_hbm, w1_hbm, out_hbm,
                  i0_vs, i1_vs, w0_v, w1_v, a_v, b_v, sem0, sem1, sems):
        wid = lax.axis_index("s") * info.num_cores + lax.axis_index("c")
        base = wid * t_per_w
        for c in range(n_ch):
            cbase = base + c * CH
            pltpu.sync_copy(pos0_hbm.at[pl.ds(cbase, CH)], i0_vs[c])
            pltpu.sync_copy(pos1_hbm.at[pl.ds(cbase, CH)], i1_vs[c])
        pltpu.sync_copy(w0_hbm.at[pl.ds(base, t_per_w)], w0_v)
        pltpu.sync_copy(w1_hbm.at[pl.ds(base, t_per_w)], w1_v)

        def gathers(c):
            return (pltpu.async_copy(yg_hbm.at[i0_vs[c]], a_v.at[c % 2],
                                     sem0[c % 2]),
                    pltpu.async_copy(yg_hbm.at[i1_vs[c]], b_v.at[c % 2],
                                     sem1[c % 2]))

        gops = {0: gathers(0)}
        sops = {}
        for c in range(n_ch):
            slot = c % 2
            for cp in gops[c]:
                cp.wait()
            if c - 1 in sops:
                sops.pop(c - 1).wait()
            if c + 1 < n_ch:
                gops[c + 1] = gathers(c + 1)

            def add_row(r, _):
                w0b = w0_v[c * CH + r, :]
                w1b = w1_v[c * CH + r, :]

                def add_vec(v, _):
                    sl = pl.ds(v * L, L)
                    a_v[slot, r, sl] = (a_v[slot, r, sl] * w0b
                                        + b_v[slot, r, sl] * w1b)
                    return 0
                return lax.fori_loop(0, H // L, add_vec, 0)

            lax.fori_loop(0, CH, add_row, 0)
            sops[c] = pltpu.async_copy(
                a_v.at[slot], out_hbm.at[pl.ds(base + c * CH, CH)],
                sems[slot])
        for c in sorted(sops):
            sops[c].wait()

    return combine_k


def _ffn_body(sched_ref, xg_ref, gw_hbm, uw_hbm, dw_hbm, out_ref,
              acc_ref, gsc, usc, dsc, gsem, usem, dsem,
              *, n_ff_blocks, fb):
    k = pl.program_id(0)
    j = sched_ref[1, k]
    t = sched_ref[2, k]
    buf = sched_ref[6, k]

    def wcopies(slot, pe, pj):
        return (
            pltpu.make_async_copy(
                gw_hbm.at[pe, pl.ds(pj * fb, fb), :], gsc.at[slot],
                gsem.at[slot]),
            pltpu.make_async_copy(
                uw_hbm.at[pe, pl.ds(pj * fb, fb), :], usc.at[slot],
                usem.at[slot]),
            pltpu.make_async_copy(
                dw_hbm.at[pe, :, pl.ds(pj * fb, fb)], dsc.at[slot],
                dsem.at[slot]),
        )

    @pl.when(sched_ref[5, k] == 1)
    def _():
        @pl.when(k == 0)
        def _():
            for cp in wcopies(buf, sched_ref[0, k], sched_ref[1, k]):
                cp.start()

        for cp in wcopies(buf, sched_ref[0, k], sched_ref[1, k]):
            cp.wait()

        @pl.when(sched_ref[7, k] == 1)
        def _():
            for cp in wcopies(1 - buf, sched_ref[8, k], sched_ref[9, k]):
                cp.start()

    @pl.when(sched_ref[4, k] == 1)
    def _():
        x = xg_ref[...]
        gate = lax.dot_general(
            x, gsc.at[buf][...], (((1,), (1,)), ((), ())),
            preferred_element_type=jnp.float32)
        up = lax.dot_general(
            x, usc.at[buf][...], (((1,), (1,)), ((), ())),
            preferred_element_type=jnp.float32)
        inter = (gate * jax.nn.sigmoid(gate)) * up
        part = lax.dot_general(
            inter, dsc.at[buf][...], (((1,), (1,)), ((), ())),
            preferred_element_type=jnp.float32)

        @pl.when(j == 0)
        def _():
            acc_ref[t] = part

        @pl.when((j > 0) & (j < n_ff_blocks - 1))
        def _():
            acc_ref[t] = acc_ref[t] + part

        @pl.when(j == n_ff_blocks - 1)
        def _():
            out_ref[...] = acc_ref[t] + part


def kernel(x, expert_indices, expert_weights, gate_proj, up_proj, down_proj):
    b, s, h = x.shape
    E, FF, _ = gate_proj.shape
    topk = expert_indices.shape[-1]
    P = b * s * topk
    G_MAX = P // T + (E - 1)
    P_MAX = G_MAX * T
    J = FF // FB

    T_MAX = -(-P // T)
    N_STEPS = G_MAX * J
    x_flat = x.reshape(b * s, h)
    tok, pos, sched = _routing_metadata(
        expert_indices, expert_weights, E, T, G_MAX, T_MAX, J)

    xg = _make_sc_dispatch(P, P_MAX, h)(x_flat, tok, pos.astype(jnp.int32))

    def out_idx(k, sc):
        ok = (sc[1, k] == J - 1) & (sc[4, k] == 1)
        return (jnp.where(ok, sc[3, k], G_MAX), 0)

    grid_spec = pltpu.PrefetchScalarGridSpec(
        num_scalar_prefetch=1,
        grid=(N_STEPS,),
        in_specs=[
            pl.BlockSpec((T, h), lambda k, sc: (sc[3, k], 0)),
            pl.BlockSpec(memory_space=pl.ANY),
            pl.BlockSpec(memory_space=pl.ANY),
            pl.BlockSpec(memory_space=pl.ANY),
        ],
        out_specs=pl.BlockSpec((T, h), out_idx),
        scratch_shapes=[
            pltpu.VMEM((T_MAX, T, h), jnp.float32),
            pltpu.VMEM((2, FB, h), jnp.float32),
            pltpu.VMEM((2, FB, h), jnp.float32),
            pltpu.VMEM((2, h, FB), jnp.float32),
            pltpu.SemaphoreType.DMA((2,)),
            pltpu.SemaphoreType.DMA((2,)),
            pltpu.SemaphoreType.DMA((2,)),
        ],
    )
    yg = pl.pallas_call(
        functools.partial(_ffn_body, n_ff_blocks=J, fb=FB),
        grid_spec=grid_spec,
        out_shape=jax.ShapeDtypeStruct(((G_MAX + 1) * T, h), jnp.float32),
        compiler_params=pltpu.CompilerParams(
            dimension_semantics=("arbitrary",)),
    )(sched, xg, gate_proj, up_proj, down_proj)

    pos2 = pos.reshape(b * s, topk)
    ew = expert_weights.reshape(b * s, topk)
    ones = jnp.ones((1, 16), jnp.float32)
    out = _make_sc_combine(b * s, h, (G_MAX + 1) * T)(
        yg, pos2[:, 0].astype(jnp.int32), pos2[:, 1].astype(jnp.int32),
        ew[:, 0:1] * ones, ew[:, 1:2] * ones)
    return out.reshape(b, s, h)


# speedup vs baseline: 1.0901x; 1.0299x over previous
import functools

import jax
import jax.numpy as jnp
from jax import lax
from jax.experimental import pallas as pl
from jax.experimental.pallas import tpu as pltpu
from jax.experimental.pallas import tpu_sc as plsc

T = 256
FB = 1024


def _routing_metadata(expert_indices, expert_weights, E, T, G_MAX, T_MAX, n_ff):
    P = expert_indices.size
    e = expert_indices.reshape(P).astype(jnp.int32)
    w = expert_weights.reshape(P)
    topk = expert_indices.shape[-1]
    tok = (jnp.arange(P, dtype=jnp.int32) // topk).astype(jnp.int32)

    onehot = (e[:, None] == jnp.arange(E, dtype=jnp.int32)[None, :]).astype(jnp.int32)
    counts = jnp.sum(onehot, axis=0)
    ranks = jnp.cumsum(onehot, axis=0) - onehot

    tiles_per_e = (counts + T - 1) // T
    tile_start = jnp.concatenate(
        [jnp.zeros((1,), jnp.int32), jnp.cumsum(tiles_per_e).astype(jnp.int32)])
    padded_start = tile_start[:-1] * T
    pos = jnp.sum(onehot * (ranks + padded_start[None, :]), axis=1)

    del w

    J = n_ff
    n_e = tiles_per_e.astype(jnp.int32)
    N_STEPS = G_MAX * J
    n_steps = tile_start[E] * J
    k_ids = jnp.arange(N_STEPS, dtype=jnp.int32)
    live = k_ids < n_steps
    kc = jnp.minimum(k_ids, jnp.maximum(n_steps - 1, 0))
    estep = J * tile_start
    sched_e = jnp.sum((kc[:, None] >= estep[1:][None, :]).astype(jnp.int32),
                      axis=1)
    ne_k = jnp.maximum(n_e[sched_e], 1)
    r = kc - estep[sched_e]
    sched_j = r // ne_k
    sched_t = r - sched_j * ne_k
    sched_g = tile_start[sched_e] + sched_t
    sched_valid = live.astype(jnp.int32)
    grp_first = jnp.concatenate(
        [jnp.ones((1,), jnp.int32),
         ((sched_e[1:] != sched_e[:-1]) | (sched_j[1:] != sched_j[:-1])
          ).astype(jnp.int32) * sched_valid[1:]])
    gid = jnp.cumsum(grp_first) - 1
    e_row = jnp.arange(E, dtype=jnp.int32)
    ne_next = jnp.min(
        jnp.where((e_row[None, :] > e_row[:, None]) & (n_e[None, :] > 0),
                  e_row[None, :], E), axis=1)
    last_j = sched_j == (J - 1)
    pf_e = jnp.where(last_j, ne_next[sched_e], sched_e)
    pf_j = jnp.where(last_j, 0, sched_j + 1)
    pf_valid = grp_first * (1 - (last_j & (ne_next[sched_e] == E)
                                 ).astype(jnp.int32))
    pf_e = jnp.minimum(pf_e, E - 1)
    cur_buf = (gid % 2).astype(jnp.int32)
    sched = jnp.stack([sched_e, sched_j, sched_t, sched_g, sched_valid,
                       grp_first, cur_buf, pf_valid, pf_e, pf_j])
    return tok, pos, sched


def _make_sc_dispatch(P, P_MAX, H):
    info = plsc.get_sparse_core_info()
    NW = info.num_cores * info.num_subcores
    b_per_w = P // NW
    L = info.num_lanes
    CHUNK = 32
    nch = b_per_w // CHUNK
    NBUF = 3
    mesh = plsc.VectorSubcoreMesh(core_axis_name="c", subcore_axis_name="s")

    @functools.partial(
        pl.kernel,
        out_type=jax.ShapeDtypeStruct((P_MAX, H), jnp.float32),
        mesh=mesh,
        scratch_types=[
            [pltpu.VMEM((CHUNK,), jnp.int32) for _ in range(nch)],
            [pltpu.VMEM((CHUNK,), jnp.int32) for _ in range(nch)],
            pltpu.VMEM((NBUF, CHUNK, H), jnp.float32),
            [pltpu.SemaphoreType.DMA] * NBUF,
            [pltpu.SemaphoreType.DMA] * NBUF,
        ],
    )
    def dispatch_k(x_hbm, tok_hbm, pos_hbm, xg_hbm,
                   tok_vs, pos_vs, rows_v, gsems, ssems):
        wid = lax.axis_index("s") * info.num_cores + lax.axis_index("c")
        base = wid * b_per_w

        def start_gather(i):
            return pltpu.async_copy(
                x_hbm.at[tok_vs[i]], rows_v.at[i % NBUF], gsems[i % NBUF])

        def start_scatter(i):
            return pltpu.async_copy(
                rows_v.at[i % NBUF], xg_hbm.at[pos_vs[i]], ssems[i % NBUF])

        for i in range(nch):
            off = base + i * CHUNK
            pltpu.sync_copy(tok_hbm.at[pl.ds(off, CHUNK)], tok_vs[i])
            pltpu.sync_copy(pos_hbm.at[pl.ds(off, CHUNK)], pos_vs[i])
        gops = {}
        sops = {}
        for i in range(min(NBUF - 1, nch)):
            gops[i] = start_gather(i)
        for i in range(nch):
            gops[i].wait()
            sops[i] = start_scatter(i)
            k = i + NBUF - 1
            if k < nch:
                if k - NBUF >= 0:
                    sops[k - NBUF].wait()
                gops[k] = start_gather(k)
        for i in range(max(0, nch - NBUF), nch):
            if i in sops:
                sops[i].wait()

    return dispatch_k


def _make_sc_combine(S, H, P_MAX):
    info = plsc.get_sparse_core_info()
    NW = info.num_cores * info.num_subcores
    t_per_w = S // NW
    CH = 24
    n_ch = -(-t_per_w // CH)
    sizes = [min(CH, t_per_w - c * CH) for c in range(n_ch)]
    offs = [c * CH for c in range(n_ch)]
    L = info.num_lanes
    mesh = plsc.VectorSubcoreMesh(core_axis_name="c", subcore_axis_name="s")

    @functools.partial(
        pl.kernel,
        out_type=jax.ShapeDtypeStruct((S, H), jnp.float32),
        mesh=mesh,
        scratch_types=[
            [pltpu.VMEM((sz,), jnp.int32) for sz in sizes],
            [pltpu.VMEM((sz,), jnp.int32) for sz in sizes],
            pltpu.VMEM((t_per_w, 16), jnp.float32),
            pltpu.VMEM((t_per_w, 16), jnp.float32),
            pltpu.VMEM((2, CH, H), jnp.float32),
            pltpu.VMEM((2, CH, H), jnp.float32),
            [pltpu.SemaphoreType.DMA] * 2,
            [pltpu.SemaphoreType.DMA] * 2,
            [pltpu.SemaphoreType.DMA] * 2,
        ],
    )
    def combine_k(yg_hbm, pos0_hbm, pos1_hbm, w0_hbm, w1_hbm, out_hbm,
                  i0_vs, i1_vs, w0_v, w1_v, a_v, b_v, sem0, sem1, sems):
        wid = lax.axis_index("s") * info.num_cores + lax.axis_index("c")
        base = wid * t_per_w
        for c in range(n_ch):
            cbase = base + offs[c]
            pltpu.sync_copy(pos0_hbm.at[pl.ds(cbase, sizes[c])], i0_vs[c])
            pltpu.sync_copy(pos1_hbm.at[pl.ds(cbase, sizes[c])], i1_vs[c])
        pltpu.sync_copy(w0_hbm.at[pl.ds(base, t_per_w)], w0_v)
        pltpu.sync_copy(w1_hbm.at[pl.ds(base, t_per_w)], w1_v)

        def gathers(c):
            return (pltpu.async_copy(yg_hbm.at[i0_vs[c]],
                                     a_v.at[c % 2, pl.ds(0, sizes[c])],
                                     sem0[c % 2]),
                    pltpu.async_copy(yg_hbm.at[i1_vs[c]],
                                     b_v.at[c % 2, pl.ds(0, sizes[c])],
                                     sem1[c % 2]))

        gops = {0: gathers(0)}
        sops = {}
        for c in range(n_ch):
            slot = c % 2
            for cp in gops[c]:
                cp.wait()
            if c - 1 in sops:
                sops.pop(c - 1).wait()
            if c + 1 < n_ch:
                gops[c + 1] = gathers(c + 1)

            def add_row(r, _):
                w0b = w0_v[offs[c] + r, :]
                w1b = w1_v[offs[c] + r, :]

                def add_vec(v, _):
                    sl = pl.ds(v * L, L)
                    a_v[slot, r, sl] = (a_v[slot, r, sl] * w0b
                                        + b_v[slot, r, sl] * w1b)
                    return 0
                return lax.fori_loop(0, H // L, add_vec, 0)

            lax.fori_loop(0, sizes[c], add_row, 0)
            sops[c] = pltpu.async_copy(
                a_v.at[slot, pl.ds(0, sizes[c])],
                out_hbm.at[pl.ds(base + offs[c], sizes[c])],
                sems[slot])
        for c in sorted(sops):
            sops[c].wait()

    return combine_k


def _ffn_body(sched_ref, xg_ref, gw_hbm, uw_hbm, dw_hbm, out_ref,
              acc_ref, gsc, usc, dsc, gsem, usem, dsem,
              *, n_ff_blocks, fb):
    k = pl.program_id(0)
    j = sched_ref[1, k]
    t = sched_ref[2, k]
    buf = sched_ref[6, k]

    def wcopies(slot, pe, pj):
        return (
            pltpu.make_async_copy(
                gw_hbm.at[pe, pl.ds(pj * fb, fb), :], gsc.at[slot],
                gsem.at[slot]),
            pltpu.make_async_copy(
                uw_hbm.at[pe, pl.ds(pj * fb, fb), :], usc.at[slot],
                usem.at[slot]),
            pltpu.make_async_copy(
                dw_hbm.at[pe, :, pl.ds(pj * fb, fb)], dsc.at[slot],
                dsem.at[slot]),
        )

    @pl.when(sched_ref[5, k] == 1)
    def _():
        @pl.when(k == 0)
        def _():
            for cp in wcopies(buf, sched_ref[0, k], sched_ref[1, k]):
                cp.start()

        for cp in wcopies(buf, sched_ref[0, k], sched_ref[1, k]):
            cp.wait()

        @pl.when(sched_ref[7, k] == 1)
        def _():
            for cp in wcopies(1 - buf, sched_ref[8, k], sched_ref[9, k]):
                cp.start()

    @pl.when(sched_ref[4, k] == 1)
    def _():
        x = xg_ref[...]
        gate = lax.dot_general(
            x, gsc.at[buf][...], (((1,), (1,)), ((), ())),
            preferred_element_type=jnp.float32)
        up = lax.dot_general(
            x, usc.at[buf][...], (((1,), (1,)), ((), ())),
            preferred_element_type=jnp.float32)
        inter = (gate * jax.nn.sigmoid(gate)) * up
        part = lax.dot_general(
            inter, dsc.at[buf][...], (((1,), (1,)), ((), ())),
            preferred_element_type=jnp.float32)

        @pl.when(j == 0)
        def _():
            acc_ref[t] = part

        @pl.when((j > 0) & (j < n_ff_blocks - 1))
        def _():
            acc_ref[t] = acc_ref[t] + part

        @pl.when(j == n_ff_blocks - 1)
        def _():
            out_ref[...] = acc_ref[t] + part


def kernel(x, expert_indices, expert_weights, gate_proj, up_proj, down_proj):
    b, s, h = x.shape
    E, FF, _ = gate_proj.shape
    topk = expert_indices.shape[-1]
    P = b * s * topk
    G_MAX = P // T + (E - 1)
    P_MAX = G_MAX * T
    J = FF // FB

    T_MAX = -(-P // T)
    N_STEPS = G_MAX * J
    x_flat = x.reshape(b * s, h)
    tok, pos, sched = _routing_metadata(
        expert_indices, expert_weights, E, T, G_MAX, T_MAX, J)

    xg = _make_sc_dispatch(P, P_MAX, h)(x_flat, tok, pos.astype(jnp.int32))

    def out_idx(k, sc):
        ok = (sc[1, k] == J - 1) & (sc[4, k] == 1)
        return (jnp.where(ok, sc[3, k], G_MAX), 0)

    grid_spec = pltpu.PrefetchScalarGridSpec(
        num_scalar_prefetch=1,
        grid=(N_STEPS,),
        in_specs=[
            pl.BlockSpec((T, h), lambda k, sc: (sc[3, k], 0)),
            pl.BlockSpec(memory_space=pl.ANY),
            pl.BlockSpec(memory_space=pl.ANY),
            pl.BlockSpec(memory_space=pl.ANY),
        ],
        out_specs=pl.BlockSpec((T, h), out_idx),
        scratch_shapes=[
            pltpu.VMEM((T_MAX, T, h), jnp.float32),
            pltpu.VMEM((2, FB, h), jnp.float32),
            pltpu.VMEM((2, FB, h), jnp.float32),
            pltpu.VMEM((2, h, FB), jnp.float32),
            pltpu.SemaphoreType.DMA((2,)),
            pltpu.SemaphoreType.DMA((2,)),
            pltpu.SemaphoreType.DMA((2,)),
        ],
    )
    yg = pl.pallas_call(
        functools.partial(_ffn_body, n_ff_blocks=J, fb=FB),
        grid_spec=grid_spec,
        out_shape=jax.ShapeDtypeStruct(((G_MAX + 1) * T, h), jnp.float32),
        compiler_params=pltpu.CompilerParams(
            dimension_semantics=("arbitrary",)),
    )(sched, xg, gate_proj, up_proj, down_proj)

    pos2 = pos.reshape(b * s, topk)
    ew = expert_weights.reshape(b * s, topk)
    ones = jnp.ones((1, 16), jnp.float32)
    out = _make_sc_combine(b * s, h, (G_MAX + 1) * T)(
        yg, pos2[:, 0].astype(jnp.int32), pos2[:, 1].astype(jnp.int32),
        ew[:, 0:1] * ones, ew[:, 1:2] * ones)
    return out.reshape(b, s, h)
